# Initial kernel scaffold; baseline (speedup 1.0000x reference)
import jax
import jax.numpy as jnp
from jax.experimental import pallas as pl


def kernel(x, adj, W1, b1, W2, b2):
    raise NotImplementedError("write your pallas kernel here")


# speedup vs baseline: 30.1242x; 30.1242x over previous
import functools

import jax
import jax.numpy as jnp
from jax import lax
from jax.experimental import pallas as pl
from jax.experimental.pallas import tpu as pltpu

_N = 1024 * 1024
_K = 734002
_ONE_BITS = 0x3F800000


def _gcn_kernel(adj_ref, x_ref, w1_ref, b1_ref, w2_ref, b2_ref, out_ref):
    b = pl.program_id(0)

    def count_le(t):
        return jnp.sum((adj_ref[0] <= t).astype(jnp.int32))

    def body(_, carry):
        lo, hi = carry
        mid = (lo + hi) // 2
        t = lax.bitcast_convert_type(mid, jnp.float32)
        pred = count_le(t) >= _K + 1
        return jnp.where(pred, lo, mid + 1), jnp.where(pred, mid, hi)

    lo, _ = lax.fori_loop(0, 31, body, (jnp.int32(0), jnp.int32(_ONE_BITS)))
    a_k = lax.bitcast_convert_type(lo, jnp.float32)

    adj = adj_ref[0]
    c_le = jnp.sum((adj <= a_k).astype(jnp.int32))
    a_next = jnp.min(jnp.where(adj > a_k, adj, 2.0))
    a_k1 = jnp.where(c_le >= _K + 2, a_k, a_next)
    thresh = 0.5 * a_k + 0.5 * a_k1

    mask = (adj_ref[0] > thresh).astype(jnp.float32)
    h0 = jnp.dot(x_ref[0], w1_ref[...], preferred_element_type=jnp.float32)
    h0 = h0 + b1_ref[...]
    h1 = jnp.maximum(jnp.dot(mask, h0, preferred_element_type=jnp.float32), 0.0)
    h2 = jnp.dot(h1, w2_ref[...], preferred_element_type=jnp.float32)
    h2 = h2 + b2_ref[...]
    h2 = jnp.maximum(jnp.dot(mask, h2, preferred_element_type=jnp.float32), 0.0)
    out_ref[pl.ds(b, 1), :] = jnp.mean(h2, axis=0, keepdims=True)


@jax.jit
def kernel(x, adj, W1, b1, W2, b2):
    bsz = adj.shape[0]
    grid = (bsz,)
    out = pl.pallas_call(
        _gcn_kernel,
        grid=grid,
        in_specs=[
            pl.BlockSpec((1, 1024, 1024), lambda b: (b, 0, 0)),
            pl.BlockSpec((1, 1024, 128), lambda b: (b, 0, 0)),
            pl.BlockSpec((128, 128), lambda b: (0, 0)),
            pl.BlockSpec((1, 128), lambda b: (0, 0)),
            pl.BlockSpec((128, 128), lambda b: (0, 0)),
            pl.BlockSpec((1, 128), lambda b: (0, 0)),
        ],
        out_specs=pl.BlockSpec((bsz, 128), lambda b: (0, 0)),
        out_shape=jax.ShapeDtypeStruct((bsz, 128), jnp.float32),
    )(adj, x, W1, b1.reshape(1, 128), W2, b2.reshape(1, 128))
    return out
